# baseline (speedup 1.0000x reference)
import jax
import jax.numpy as jnp
from jax.experimental import pallas as pl
from jax.experimental.pallas import tpu as pltpu

_N = 10000
_D = 128
_OUT = 128
_BM = 200


def _sage_fused_kernel(a_ref, f_all_ref, f_blk_ref, wt_ref, out_ref):
    a = a_ref[...]
    neigh = jnp.dot(a, f_all_ref[...],
                    preferred_element_type=jnp.float32)
    rowsum = jnp.sum(a, axis=1, keepdims=True)
    neigh = neigh / (rowsum + 1.0)
    out = jnp.dot(f_blk_ref[...], wt_ref[:_D, :],
                  preferred_element_type=jnp.float32)
    out = out + jnp.dot(neigh, wt_ref[_D:, :],
                        preferred_element_type=jnp.float32)
    out_ref[...] = out


def kernel(adj, features, W):
    wt = W.T
    grid = _N // _BM
    return pl.pallas_call(
        _sage_fused_kernel,
        grid=(grid,),
        in_specs=[
            pl.BlockSpec((_BM, _N), lambda i: (i, 0)),
            pl.BlockSpec((_N, _D), lambda i: (0, 0)),
            pl.BlockSpec((_BM, _D), lambda i: (i, 0)),
            pl.BlockSpec((2 * _D, _OUT), lambda i: (0, 0)),
        ],
        out_specs=pl.BlockSpec((_BM, _OUT), lambda i: (i, 0)),
        out_shape=jax.ShapeDtypeStruct((_N, _OUT), jnp.float32),
        compiler_params=pltpu.CompilerParams(
            dimension_semantics=("parallel",),
        ),
    )(adj, features, features, wt)


# speedup vs baseline: 1.0319x; 1.0319x over previous
import jax
import jax.numpy as jnp
from jax.experimental import pallas as pl
from jax.experimental.pallas import tpu as pltpu

_N = 10000
_D = 128
_OUT = 128
_BM = 400


def _sage_fused_kernel(a_ref, f_all_ref, f_blk_ref, wt_ref, out_ref):
    a = a_ref[...]
    neigh = jnp.dot(a, f_all_ref[...],
                    preferred_element_type=jnp.float32)
    rowsum = jnp.sum(a, axis=1, keepdims=True)
    neigh = neigh / (rowsum + 1.0)
    out = jnp.dot(f_blk_ref[...], wt_ref[:_D, :],
                  preferred_element_type=jnp.float32)
    out = out + jnp.dot(neigh, wt_ref[_D:, :],
                        preferred_element_type=jnp.float32)
    out_ref[...] = out


def kernel(adj, features, W):
    wt = W.T
    grid = _N // _BM
    return pl.pallas_call(
        _sage_fused_kernel,
        grid=(grid,),
        in_specs=[
            pl.BlockSpec((_BM, _N), lambda i: (i, 0)),
            pl.BlockSpec((_N, _D), lambda i: (0, 0)),
            pl.BlockSpec((_BM, _D), lambda i: (i, 0)),
            pl.BlockSpec((2 * _D, _OUT), lambda i: (0, 0)),
        ],
        out_specs=pl.BlockSpec((_BM, _OUT), lambda i: (i, 0)),
        out_shape=jax.ShapeDtypeStruct((_N, _OUT), jnp.float32),
        compiler_params=pltpu.CompilerParams(
            dimension_semantics=("parallel",),
        ),
    )(adj, features, features, wt)
